# Initial kernel scaffold; baseline (speedup 1.0000x reference)
import functools

import jax
import jax.numpy as jnp
from jax import lax
from jax.experimental import pallas as pl
from jax.experimental.pallas import tpu as pltpu
from jax.experimental.pallas import tpu_sc as plsc

F32 = jnp.float32
I32 = jnp.int32

NC = 2
TPS = 16
NW = NC * TPS
CH = 128
BLK = 1280


def _rnorm_tc(xp):
    NP, D = xp.shape

    def body(x_ref, rn_ref):
        xs = x_ref[...]
        ss = jnp.sum(xs * xs, axis=1)
        rn_ref[...] = (1.0 / jnp.maximum(jnp.sqrt(ss), 1e-12))[None, :]

    return pl.pallas_call(
        body,
        grid=(NP // BLK,),
        in_specs=[pl.BlockSpec((BLK, D), lambda i: (i, 0))],
        out_specs=pl.BlockSpec((1, BLK), lambda i: (i, 0)),
        out_shape=jax.ShapeDtypeStruct((NP // BLK, BLK), F32),
    )(xp)


def _rdenom_tc(dp):
    _, NP = dp.shape

    def body(d_ref, rd_ref):
        t = d_ref[0, 0, :] + d_ref[1, 0, :]
        rd_ref[...] = (1.0 / jnp.maximum(t, 1e-16))[None, :]

    dp3 = dp.reshape(2, NP // BLK, BLK)
    return pl.pallas_call(
        body,
        grid=(NP // BLK,),
        in_specs=[pl.BlockSpec((2, 1, BLK), lambda i: (0, i, 0))],
        out_specs=pl.BlockSpec((1, BLK), lambda i: (i, 0)),
        out_shape=jax.ShapeDtypeStruct((NP // BLK, BLK), F32),
    )(dp3)


def _combine_tc(op):
    _, NP, D = op.shape

    def body(p_ref, h_ref, rn_ref):
        v = p_ref[0] + p_ref[1]
        h = jnp.maximum(v, 0.0)
        h_ref[...] = h
        ss = jnp.sum(h * h, axis=1)
        rn_ref[...] = (1.0 / jnp.maximum(jnp.sqrt(ss), 1e-12))[None, :]

    return pl.pallas_call(
        body,
        grid=(NP // BLK,),
        in_specs=[pl.BlockSpec((2, BLK, D), lambda i: (0, i, 0))],
        out_specs=[
            pl.BlockSpec((BLK, D), lambda i: (i, 0)),
            pl.BlockSpec((1, BLK), lambda i: (i, 0)),
        ],
        out_shape=[
            jax.ShapeDtypeStruct((NP, D), F32),
            jax.ShapeDtypeStruct((NP // BLK, BLK), F32),
        ],
    )(op)


def _sc_pass1(xp, rnorm, bp, src, dst):
    NP, D = xp.shape
    Et_pad = src.shape[0]
    per_tile = Et_pad // NW
    n_chunks = per_tile // CH
    sl = NP // TPS
    mesh = plsc.VectorSubcoreMesh(core_axis_name="c", subcore_axis_name="s")

    def body(x_hbm, rn_hbm, bp_hbm, src_hbm, dst_hbm, w_hbm, dnm_hbm,
             rn_v, bp_v, srci, dsti, rows_s, rows_d, dots_v, w_v,
             dnm_sh, sem1, sem2):
        c = lax.axis_index("c")
        s = lax.axis_index("s")
        wid = s * NC + c
        pltpu.sync_copy(rn_hbm, rn_v)
        pltpu.sync_copy(bp_hbm, bp_v)
        betav = bp_v[0]
        shiftv = bp_v[1]

        def zg(g, _):
            dots_v[pl.ds(g * 16, 16)] = jnp.zeros((16,), F32)
            return 0
        lax.fori_loop(0, CH // 16, zg, 0)

        def zc(j, _):
            pltpu.sync_copy(dots_v, dnm_sh.at[pl.ds(s * sl + j * CH, CH)])
            return 0
        lax.fori_loop(0, sl // CH, zc, 0)
        plsc.subcore_barrier()

        def chunk(i, _):
            off = wid * per_tile + i * CH
            pltpu.sync_copy(src_hbm.at[pl.ds(off, CH)], srci)
            pltpu.sync_copy(dst_hbm.at[pl.ds(off, CH)], dsti)
            g1 = pltpu.async_copy(x_hbm.at[srci], rows_s, sem1)
            g2 = pltpu.async_copy(x_hbm.at[dsti], rows_d, sem2)
            g1.wait()
            g2.wait()

            def edge(e, _):
                acc = rows_s[e, pl.ds(0, 16)] * rows_d[e, pl.ds(0, 16)]
                for k in range(1, D // 16):
                    acc = acc + (rows_s[e, pl.ds(k * 16, 16)]
                                 * rows_d[e, pl.ds(k * 16, 16)])
                dots_v[e] = jnp.sum(acc)
                return 0
            lax.fori_loop(0, CH, edge, 0)

            def grp(g, _):
                ids_s = srci[pl.ds(g * 16, 16)]
                ids_d = dsti[pl.ds(g * 16, 16)]
                rs = plsc.load_gather(rn_v, [ids_s])
                rd = plsc.load_gather(rn_v, [ids_d])
                dv = dots_v[pl.ds(g * 16, 16)]
                w_v[pl.ds(g * 16, 16)] = jnp.exp(dv * rs * rd * betav - shiftv)
                return 0
            lax.fori_loop(0, CH // 16, grp, 0)

            pltpu.sync_copy(w_v, w_hbm.at[pl.ds(off, CH)])
            pltpu.sync_copy(w_v, dnm_sh.at[dsti], add=True)
            return 0
        lax.fori_loop(0, n_chunks, chunk, 0)

        plsc.subcore_barrier()
        pltpu.sync_copy(dnm_sh.at[pl.ds(s * sl, sl)],
                        dnm_hbm.at[c, pl.ds(s * sl, sl)])

    f = pl.kernel(
        body,
        out_type=(
            jax.ShapeDtypeStruct((Et_pad,), F32),
            jax.ShapeDtypeStruct((NC, NP), F32),
        ),
        mesh=mesh,
        scratch_types=[
            pltpu.VMEM((NP,), F32),
            pltpu.VMEM((2, 16), F32),
            pltpu.VMEM((CH,), I32),
            pltpu.VMEM((CH,), I32),
            pltpu.VMEM((CH, D), F32),
            pltpu.VMEM((CH, D), F32),
            pltpu.VMEM((CH,), F32),
            pltpu.VMEM((CH,), F32),
            pltpu.VMEM_SHARED((NP,), F32),
            pltpu.SemaphoreType.DMA,
            pltpu.SemaphoreType.DMA,
        ],
    )
    return f(xp, rnorm, bp, src, dst)


def _sc_pass2(xp, rdenom, w, src, dst):
    NP, D = xp.shape
    Et_pad = src.shape[0]
    per_tile = Et_pad // NW
    n_chunks = per_tile // CH
    sl = NP // TPS
    mesh = plsc.VectorSubcoreMesh(core_axis_name="c", subcore_axis_name="s")

    def body(x_hbm, rd_hbm, w_hbm, src_hbm, dst_hbm, out_hbm,
             rd_v, srci, dsti, w_v, coef_v, rows, out_sh, sem1):
        c = lax.axis_index("c")
        s = lax.axis_index("s")
        wid = s * NC + c
        pltpu.sync_copy(rd_hbm, rd_v)

        def zrow(e, _):
            def zk(k, _):
                rows[e, pl.ds(k * 16, 16)] = jnp.zeros((16,), F32)
                return 0
            lax.fori_loop(0, D // 16, zk, 0)
            return 0
        lax.fori_loop(0, CH, zrow, 0)

        def zc(j, _):
            pltpu.sync_copy(rows, out_sh.at[pl.ds(s * sl + j * CH, CH)])
            return 0
        lax.fori_loop(0, sl // CH, zc, 0)
        plsc.subcore_barrier()

        def chunk(i, _):
            off = wid * per_tile + i * CH
            pltpu.sync_copy(src_hbm.at[pl.ds(off, CH)], srci)
            pltpu.sync_copy(dst_hbm.at[pl.ds(off, CH)], dsti)
            pltpu.sync_copy(w_hbm.at[pl.ds(off, CH)], w_v)
            pltpu.async_copy(x_hbm.at[srci], rows, sem1).wait()

            def grp(g, _):
                ids_d = dsti[pl.ds(g * 16, 16)]
                rd = plsc.load_gather(rd_v, [ids_d])
                coef_v[pl.ds(g * 16, 16)] = w_v[pl.ds(g * 16, 16)] * rd
                return 0
            lax.fori_loop(0, CH // 16, grp, 0)

            def edge(e, _):
                cst = coef_v[e]
                for k in range(D // 16):
                    rows[e, pl.ds(k * 16, 16)] = rows[e, pl.ds(k * 16, 16)] * cst
                return 0
            lax.fori_loop(0, CH, edge, 0)

            pltpu.sync_copy(rows, out_sh.at[dsti], add=True)
            return 0
        lax.fori_loop(0, n_chunks, chunk, 0)

        plsc.subcore_barrier()
        pltpu.sync_copy(out_sh.at[pl.ds(s * sl, sl)],
                        out_hbm.at[c, pl.ds(s * sl, sl)])

    f = pl.kernel(
        body,
        out_type=jax.ShapeDtypeStruct((NC, NP, D), F32),
        mesh=mesh,
        scratch_types=[
            pltpu.VMEM((NP,), F32),
            pltpu.VMEM((CH,), I32),
            pltpu.VMEM((CH,), I32),
            pltpu.VMEM((CH,), F32),
            pltpu.VMEM((CH,), F32),
            pltpu.VMEM((CH, D), F32),
            pltpu.VMEM_SHARED((NP, D), F32),
            pltpu.SemaphoreType.DMA,
        ],
    )
    return f(xp, rdenom, w, src, dst)


def _layer(xp, rn, src, dst, beta):
    NP = xp.shape[0]
    bp = jnp.stack([jnp.full((16,), beta, F32),
                    jnp.full((16,), jnp.abs(beta), F32)])
    w, dp = _sc_pass1(xp, rn, bp, src, dst)
    rd = _rdenom_tc(dp).reshape(NP)
    op = _sc_pass2(xp, rd, w, src, dst)
    h, rn_next = _combine_tc(op)
    return h, rn_next.reshape(NP)


def kernel(x, edge_index, beta1, beta2):
    N, D = x.shape
    E = edge_index.shape[1]
    NP = ((N + BLK - 1) // BLK) * BLK
    Et = E + N
    per_tile = -(-Et // (NW * CH)) * CH
    Et_pad = per_tile * NW

    loops = jnp.arange(N, dtype=I32)
    pad_e = Et_pad - Et
    pad_idx = jnp.full((pad_e,), N, I32)
    src = jnp.concatenate([edge_index[0].astype(I32), loops, pad_idx])
    dst = jnp.concatenate([edge_index[1].astype(I32), loops, pad_idx])
    xp = jnp.zeros((NP, D), F32).at[:N].set(x)

    rn1 = _rnorm_tc(xp).reshape(NP)
    h1, rn2 = _layer(xp, rn1, src, dst, beta1.astype(F32))
    h2, _ = _layer(h1, rn2, src, dst, beta2.astype(F32))
    return h2[:N]


# speedup vs baseline: 3.5214x; 3.5214x over previous
import functools

import jax
import jax.numpy as jnp
from jax import lax
from jax.experimental import pallas as pl
from jax.experimental.pallas import tpu as pltpu
from jax.experimental.pallas import tpu_sc as plsc

F32 = jnp.float32
I32 = jnp.int32

NC = 2
TPS = 16
NW = NC * TPS
CH = 128
BLK = 1280


def _rnorm_tc(xp):
    NP, D = xp.shape

    def body(x_ref, rn_ref):
        xs = x_ref[...]
        ss = jnp.sum(xs * xs, axis=1)
        rn_ref[...] = (1.0 / jnp.maximum(jnp.sqrt(ss), 1e-12)).reshape(
            1, 8, BLK // 8)

    return pl.pallas_call(
        body,
        grid=(NP // BLK,),
        in_specs=[pl.BlockSpec((BLK, D), lambda i: (i, 0))],
        out_specs=pl.BlockSpec((1, 8, BLK // 8), lambda i: (i, 0, 0)),
        out_shape=jax.ShapeDtypeStruct((NP // BLK, 8, BLK // 8), F32),
    )(xp)


def _rdenom_tc(dp):
    _, NP = dp.shape

    def body(d_ref, rd_ref):
        t = d_ref[0, 0] + d_ref[1, 0]
        rd_ref[...] = (1.0 / jnp.maximum(t, 1e-16))[None]

    dp4 = dp.reshape(2, NP // BLK, 8, BLK // 8)
    return pl.pallas_call(
        body,
        grid=(NP // BLK,),
        in_specs=[pl.BlockSpec((2, 1, 8, BLK // 8), lambda i: (0, i, 0, 0))],
        out_specs=pl.BlockSpec((1, 8, BLK // 8), lambda i: (i, 0, 0)),
        out_shape=jax.ShapeDtypeStruct((NP // BLK, 8, BLK // 8), F32),
    )(dp4)


def _combine_tc(op):
    _, NP, D = op.shape

    def body(p_ref, h_ref, rn_ref):
        v = p_ref[0] + p_ref[1]
        h = jnp.maximum(v, 0.0)
        h_ref[...] = h
        ss = jnp.sum(h * h, axis=1)
        rn_ref[...] = (1.0 / jnp.maximum(jnp.sqrt(ss), 1e-12)).reshape(
            1, 8, BLK // 8)

    return pl.pallas_call(
        body,
        grid=(NP // BLK,),
        in_specs=[pl.BlockSpec((2, BLK, D), lambda i: (0, i, 0))],
        out_specs=[
            pl.BlockSpec((BLK, D), lambda i: (i, 0)),
            pl.BlockSpec((1, 8, BLK // 8), lambda i: (i, 0, 0)),
        ],
        out_shape=[
            jax.ShapeDtypeStruct((NP, D), F32),
            jax.ShapeDtypeStruct((NP // BLK, 8, BLK // 8), F32),
        ],
    )(op)


def _sc_pass1(xp, rnorm, bp, src, dst):
    NP, D = xp.shape
    Et_pad = src.shape[0]
    per_tile = Et_pad // NW
    n_chunks = per_tile // CH
    sl = NP // TPS
    mesh = plsc.VectorSubcoreMesh(core_axis_name="c", subcore_axis_name="s")

    def body(x_hbm, rn_hbm, bp_hbm, src_hbm, dst_hbm, w_hbm, dnm_hbm,
             rn_v, bp_v, srci, dsti, rows_s, rows_d, w_v,
             dnm_sh, sem1, sem2):
        c = lax.axis_index("c")
        s = lax.axis_index("s")
        wid = s * NC + c
        pltpu.sync_copy(rn_hbm, rn_v)
        pltpu.sync_copy(bp_hbm, bp_v)
        betav = bp_v[0]
        shiftv = bp_v[1]

        def zg(g, _):
            w_v[pl.ds(g * 16, 16)] = jnp.zeros((16,), F32)
            return 0
        lax.fori_loop(0, CH // 16, zg, 0)

        def zc(j, _):
            pltpu.sync_copy(w_v, dnm_sh.at[pl.ds(s * sl + j * CH, CH)])
            return 0
        lax.fori_loop(0, sl // CH, zc, 0)
        plsc.subcore_barrier()

        def chunk(i, _):
            off = wid * per_tile + i * CH
            pltpu.sync_copy(src_hbm.at[pl.ds(off, CH)], srci)
            pltpu.sync_copy(dst_hbm.at[pl.ds(off, CH)], dsti)
            g1 = pltpu.async_copy(x_hbm.at[srci], rows_s, sem1)
            g2 = pltpu.async_copy(x_hbm.at[dsti], rows_d, sem2)
            g1.wait()
            g2.wait()

            lanes16 = lax.broadcasted_iota(I32, (16,), 0)

            def grp(g, _):
                eids = lanes16 + g * 16

                def kstep(k, acc):
                    kv = jnp.broadcast_to(k, (16,))
                    sv = plsc.load_gather(rows_s, [eids, kv])
                    dv = plsc.load_gather(rows_d, [eids, kv])
                    return acc + sv * dv
                dots = lax.fori_loop(0, D, kstep, jnp.zeros((16,), F32))
                ids_s = srci[pl.ds(g * 16, 16)]
                ids_d = dsti[pl.ds(g * 16, 16)]
                rs = plsc.load_gather(rn_v, [ids_s])
                rd = plsc.load_gather(rn_v, [ids_d])
                w_v[pl.ds(g * 16, 16)] = jnp.exp(
                    dots * rs * rd * betav - shiftv)
                return 0
            lax.fori_loop(0, CH // 16, grp, 0)

            pltpu.sync_copy(w_v, w_hbm.at[pl.ds(off, CH)])
            pltpu.sync_copy(w_v, dnm_sh.at[dsti], add=True)
            return 0
        lax.fori_loop(0, n_chunks, chunk, 0)

        plsc.subcore_barrier()
        pltpu.sync_copy(dnm_sh.at[pl.ds(s * sl, sl)],
                        dnm_hbm.at[c, pl.ds(s * sl, sl)])

    f = pl.kernel(
        body,
        out_type=(
            jax.ShapeDtypeStruct((Et_pad,), F32),
            jax.ShapeDtypeStruct((NC, NP), F32),
        ),
        mesh=mesh,
        compiler_params=pltpu.CompilerParams(needs_layout_passes=False),
        scratch_types=[
            pltpu.VMEM((NP,), F32),
            pltpu.VMEM((2, 16), F32),
            pltpu.VMEM((CH,), I32),
            pltpu.VMEM((CH,), I32),
            pltpu.VMEM((CH, D), F32),
            pltpu.VMEM((CH, D), F32),
            pltpu.VMEM((CH,), F32),
            pltpu.VMEM_SHARED((NP,), F32),
            pltpu.SemaphoreType.DMA,
            pltpu.SemaphoreType.DMA,
        ],
    )
    return f(xp, rnorm, bp, src, dst)


def _sc_pass2(xp, rdenom, w, src, dst):
    NP, D = xp.shape
    Et_pad = src.shape[0]
    per_tile = Et_pad // NW
    n_chunks = per_tile // CH
    sl = NP // TPS
    mesh = plsc.VectorSubcoreMesh(core_axis_name="c", subcore_axis_name="s")

    def body(x_hbm, rd_hbm, w_hbm, src_hbm, dst_hbm, out_hbm,
             rd_v, srci, dsti, w_v, rows, out_sh, sem1):
        c = lax.axis_index("c")
        s = lax.axis_index("s")
        wid = s * NC + c
        pltpu.sync_copy(rd_hbm, rd_v)

        def zrow(e, _):
            def zk(k, _):
                rows[e, pl.ds(k * 16, 16)] = jnp.zeros((16,), F32)
                return 0
            lax.fori_loop(0, D // 16, zk, 0)
            return 0
        lax.fori_loop(0, CH, zrow, 0)

        def zc(j, _):
            pltpu.sync_copy(rows, out_sh.at[pl.ds(s * sl + j * CH, CH)])
            return 0
        lax.fori_loop(0, sl // CH, zc, 0)
        plsc.subcore_barrier()

        def chunk(i, _):
            off = wid * per_tile + i * CH
            pltpu.sync_copy(src_hbm.at[pl.ds(off, CH)], srci)
            pltpu.sync_copy(dst_hbm.at[pl.ds(off, CH)], dsti)
            pltpu.sync_copy(w_hbm.at[pl.ds(off, CH)], w_v)
            pltpu.async_copy(x_hbm.at[srci], rows, sem1).wait()

            def grp(g, _):
                ids_d = dsti[pl.ds(g * 16, 16)]
                rd = plsc.load_gather(rd_v, [ids_d])
                cvec = w_v[pl.ds(g * 16, 16)] * rd
                for l in range(16):
                    cst = cvec[l]
                    e = g * 16 + l
                    for k in range(D // 16):
                        rows[e, pl.ds(k * 16, 16)] = (
                            rows[e, pl.ds(k * 16, 16)] * cst)
                return 0
            lax.fori_loop(0, CH // 16, grp, 0)

            pltpu.sync_copy(rows, out_sh.at[dsti], add=True)
            return 0
        lax.fori_loop(0, n_chunks, chunk, 0)

        plsc.subcore_barrier()
        pltpu.sync_copy(out_sh.at[pl.ds(s * sl, sl)],
                        out_hbm.at[c, pl.ds(s * sl, sl)])

    f = pl.kernel(
        body,
        out_type=jax.ShapeDtypeStruct((NC, NP, D), F32),
        mesh=mesh,
        compiler_params=pltpu.CompilerParams(needs_layout_passes=False),
        scratch_types=[
            pltpu.VMEM((NP,), F32),
            pltpu.VMEM((CH,), I32),
            pltpu.VMEM((CH,), I32),
            pltpu.VMEM((CH,), F32),
            pltpu.VMEM((CH, D), F32),
            pltpu.VMEM_SHARED((NP, D), F32),
            pltpu.SemaphoreType.DMA,
        ],
    )
    return f(xp, rdenom, w, src, dst)


def _layer(xp, rn, src, dst, beta):
    NP = xp.shape[0]
    bp = jnp.stack([jnp.full((16,), beta, F32),
                    jnp.full((16,), jnp.abs(beta), F32)])
    w, dp = _sc_pass1(xp, rn, bp, src, dst)
    rd = _rdenom_tc(dp).reshape(NP)
    op = _sc_pass2(xp, rd, w, src, dst)
    h, rn_next = _combine_tc(op)
    return h, rn_next.reshape(NP)


def kernel(x, edge_index, beta1, beta2):
    N, D = x.shape
    E = edge_index.shape[1]
    NP = ((N + BLK - 1) // BLK) * BLK
    Et = E + N
    per_tile = -(-Et // (NW * CH)) * CH
    Et_pad = per_tile * NW

    loops = jnp.arange(N, dtype=I32)
    pad_e = Et_pad - Et
    pad_idx = jnp.full((pad_e,), N, I32)
    src = jnp.concatenate([edge_index[0].astype(I32), loops, pad_idx])
    dst = jnp.concatenate([edge_index[1].astype(I32), loops, pad_idx])
    xp = jnp.zeros((NP, D), F32).at[:N].set(x)

    rn1 = _rnorm_tc(xp).reshape(NP)
    h1, rn2 = _layer(xp, rn1, src, dst, beta1.astype(F32))
    h2, _ = _layer(h1, rn2, src, dst, beta2.astype(F32))
    return h2[:N]


# speedup vs baseline: 3.5996x; 1.0222x over previous
import functools

import jax
import jax.numpy as jnp
from jax import lax
from jax.experimental import pallas as pl
from jax.experimental.pallas import tpu as pltpu
from jax.experimental.pallas import tpu_sc as plsc

F32 = jnp.float32
I32 = jnp.int32

NC = 2
TPS = 16
NW = NC * TPS
CH = 128
BLK = 1280


def _rnorm_tc(xp):
    NP, D = xp.shape

    def body(x_ref, rn_ref):
        xs = x_ref[...]
        ss = jnp.sum(xs * xs, axis=1)
        rn_ref[...] = (1.0 / jnp.maximum(jnp.sqrt(ss), 1e-12)).reshape(
            1, 8, BLK // 8)

    return pl.pallas_call(
        body,
        grid=(NP // BLK,),
        in_specs=[pl.BlockSpec((BLK, D), lambda i: (i, 0))],
        out_specs=pl.BlockSpec((1, 8, BLK // 8), lambda i: (i, 0, 0)),
        out_shape=jax.ShapeDtypeStruct((NP // BLK, 8, BLK // 8), F32),
    )(xp)


def _combine_tc(op, dp):
    _, NP, D = op.shape

    def body(p_ref, d_ref, h_ref, rn_ref):
        v = p_ref[0] + p_ref[1]
        dn = d_ref[0] + d_ref[1]
        rdn = 1.0 / jnp.maximum(dn, 1e-16)
        h = jnp.maximum(v * rdn[:, None], 0.0)
        h_ref[...] = h
        ss = jnp.sum(h * h, axis=1)
        rn_ref[...] = (1.0 / jnp.maximum(jnp.sqrt(ss), 1e-12)).reshape(
            1, 8, BLK // 8)

    return pl.pallas_call(
        body,
        grid=(NP // BLK,),
        in_specs=[
            pl.BlockSpec((2, BLK, D), lambda i: (0, i, 0)),
            pl.BlockSpec((2, BLK), lambda i: (0, i)),
        ],
        out_specs=[
            pl.BlockSpec((BLK, D), lambda i: (i, 0)),
            pl.BlockSpec((1, 8, BLK // 8), lambda i: (i, 0, 0)),
        ],
        out_shape=[
            jax.ShapeDtypeStruct((NP, D), F32),
            jax.ShapeDtypeStruct((NP // BLK, 8, BLK // 8), F32),
        ],
    )(op, dp)


def _sc_pass1(xp, rnorm, bp, src2, dst2):
    NP, D = xp.shape
    n_chunks = src2.shape[1]
    sl = NP // TPS
    mesh = plsc.VectorSubcoreMesh(core_axis_name="c", subcore_axis_name="s")

    def body(x_hbm, rn_hbm, bp_hbm, src_hbm, dst_hbm, w_hbm, dnm_hbm,
             rn_v, bp_v, si_all, di_all, rows_s0, rows_d0, rows_s1, rows_d1,
             wt_v, zb_v, dnm_sh, sem_s0, sem_d0, sem_s1, sem_d1, sem_n):
        c = lax.axis_index("c")
        s = lax.axis_index("s")
        wid = s * NC + c
        pltpu.sync_copy(rn_hbm, rn_v)
        pltpu.sync_copy(bp_hbm, bp_v)
        pltpu.sync_copy(src_hbm.at[wid], si_all)
        pltpu.sync_copy(dst_hbm.at[wid], di_all)
        betav = bp_v[0]
        shiftv = bp_v[1]

        def zg(g, _):
            zb_v[pl.ds(g * 16, 16)] = jnp.zeros((16,), F32)
            return 0
        lax.fori_loop(0, CH // 16, zg, 0)

        def zc(j, _):
            pltpu.sync_copy(zb_v, dnm_sh.at[pl.ds(s * sl + j * CH, CH)])
            return 0
        lax.fori_loop(0, sl // CH, zc, 0)
        plsc.subcore_barrier()

        lanes16 = lax.broadcasted_iota(I32, (16,), 0)

        def issue(j, rows_s, rows_d, sem_s, sem_d):
            pltpu.async_copy(x_hbm.at[si_all.at[j]], rows_s, sem_s)
            pltpu.async_copy(x_hbm.at[di_all.at[j]], rows_d, sem_d)

        def wait(j, rows_s, rows_d, sem_s, sem_d):
            pltpu.make_async_copy(x_hbm.at[si_all.at[j]], rows_s, sem_s).wait()
            pltpu.make_async_copy(x_hbm.at[di_all.at[j]], rows_d, sem_d).wait()

        def compute(j, rows_s, rows_d):
            def grp(g, _):
                wt = jnp.zeros((16,), F32)
                for l in range(16):
                    e = g * 16 + l
                    acc = rows_s[e, pl.ds(0, 16)] * rows_d[e, pl.ds(0, 16)]
                    for k in range(1, D // 16):
                        acc = acc + (rows_s[e, pl.ds(k * 16, 16)] *
                                     rows_d[e, pl.ds(k * 16, 16)])
                    wt = wt + jnp.where(lanes16 == l, jnp.sum(acc), 0.0)
                rs = plsc.load_gather(rn_v, [si_all[j, pl.ds(g * 16, 16)]])
                rd = plsc.load_gather(rn_v, [di_all[j, pl.ds(g * 16, 16)]])
                wt_v[j, pl.ds(g * 16, 16)] = jnp.exp(
                    wt * rs * rd * betav - shiftv)
                return 0
            lax.fori_loop(0, CH // 16, grp, 0)
            pltpu.async_copy(wt_v.at[j], dnm_sh.at[di_all.at[j]], sem_n,
                             add=True)

            @pl.when(j >= 2)
            def _():
                pltpu.make_async_copy(wt_v.at[0], dnm_sh.at[di_all.at[0]],
                                      sem_n).wait()

        issue(0, rows_s0, rows_d0, sem_s0, sem_d0)
        issue(1, rows_s1, rows_d1, sem_s1, sem_d1)

        def pair(i, _):
            j0 = 2 * i
            j1 = 2 * i + 1
            wait(j0, rows_s0, rows_d0, sem_s0, sem_d0)
            compute(j0, rows_s0, rows_d0)

            @pl.when(j0 + 2 < n_chunks)
            def _():
                issue(j0 + 2, rows_s0, rows_d0, sem_s0, sem_d0)
            wait(j1, rows_s1, rows_d1, sem_s1, sem_d1)
            compute(j1, rows_s1, rows_d1)

            @pl.when(j1 + 2 < n_chunks)
            def _():
                issue(j1 + 2, rows_s1, rows_d1, sem_s1, sem_d1)
            return 0
        lax.fori_loop(0, n_chunks // 2, pair, 0)

        pltpu.sync_copy(wt_v, w_hbm.at[wid])
        pltpu.make_async_copy(wt_v.at[0], dnm_sh.at[di_all.at[0]],
                              sem_n).wait()
        pltpu.make_async_copy(wt_v.at[0], dnm_sh.at[di_all.at[0]],
                              sem_n).wait()
        plsc.subcore_barrier()
        pltpu.sync_copy(dnm_sh.at[pl.ds(s * sl, sl)],
                        dnm_hbm.at[c, pl.ds(s * sl, sl)])

    f = pl.kernel(
        body,
        out_type=(
            jax.ShapeDtypeStruct((NW, n_chunks, CH), F32),
            jax.ShapeDtypeStruct((NC, NP), F32),
        ),
        mesh=mesh,
        compiler_params=pltpu.CompilerParams(needs_layout_passes=False),
        scratch_types=[
            pltpu.VMEM((NP,), F32),
            pltpu.VMEM((2, 16), F32),
            pltpu.VMEM((n_chunks, CH), I32),
            pltpu.VMEM((n_chunks, CH), I32),
            pltpu.VMEM((CH, D), F32),
            pltpu.VMEM((CH, D), F32),
            pltpu.VMEM((CH, D), F32),
            pltpu.VMEM((CH, D), F32),
            pltpu.VMEM((n_chunks, CH), F32),
            pltpu.VMEM((CH,), F32),
            pltpu.VMEM_SHARED((NP,), F32),
            pltpu.SemaphoreType.DMA,
            pltpu.SemaphoreType.DMA,
            pltpu.SemaphoreType.DMA,
            pltpu.SemaphoreType.DMA,
            pltpu.SemaphoreType.DMA,
        ],
    )
    return f(xp, rnorm, bp, src2, dst2)


def _sc_pass2(xp, w2, src2, dst2):
    NP, D = xp.shape
    nrow = src2.shape[1]
    C2 = 32
    UN = 12
    n_chunks = nrow * 128 // C2
    n_iters = n_chunks // UN
    rpi = UN * C2 // 128
    sl = NP // TPS
    mesh = plsc.VectorSubcoreMesh(core_axis_name="c", subcore_axis_name="s")

    def body(x_hbm, w_hbm, src_hbm, dst_hbm, out_hbm,
             si_all, di_all, wt_all, rows0, rows1, rows2, out_sh,
             sem_g0, sem_g1, sem_g2, sem_c0, sem_c1, sem_c2):
        c = lax.axis_index("c")
        s = lax.axis_index("s")
        wid = s * NC + c
        pltpu.sync_copy(src_hbm.at[wid], si_all)
        pltpu.sync_copy(dst_hbm.at[wid], di_all)
        pltpu.sync_copy(w_hbm.at[wid], wt_all)

        def zrow(e, _):
            def zk(k, _):
                rows0[e, pl.ds(k * 16, 16)] = jnp.zeros((16,), F32)
                return 0
            lax.fori_loop(0, D // 16, zk, 0)
            return 0
        lax.fori_loop(0, C2, zrow, 0)

        def zc(j, _):
            pltpu.sync_copy(rows0, out_sh.at[pl.ds(s * sl + j * C2, C2)])
            return 0
        lax.fori_loop(0, sl // C2, zc, 0)
        plsc.subcore_barrier()

        bufs = ((rows0, sem_g0, sem_c0),
                (rows1, sem_g1, sem_c1),
                (rows2, sem_g2, sem_c2))

        def lst(tab, i, u):
            jj = i * rpi + (u * C2) // 128
            return tab.at[jj, pl.ds((u * C2) % 128, C2)]

        def compute(i, u, rows):
            jj = i * rpi + (u * C2) // 128
            o = (u * C2) % 128

            def grp(g, _):
                cvec = wt_all[jj, pl.ds(o + g * 16, 16)]
                for l in range(16):
                    cst = cvec[l]
                    e = g * 16 + l
                    for k in range(D // 16):
                        rows[e, pl.ds(k * 16, 16)] = (
                            rows[e, pl.ds(k * 16, 16)] * cst)
                return 0
            lax.fori_loop(0, C2 // 16, grp, 0)

        pltpu.async_copy(x_hbm.at[lst(si_all, 0, 0)], rows0, sem_g0)
        pltpu.async_copy(x_hbm.at[lst(si_all, 0, 1)], rows1, sem_g1)

        def it(i, _):
            for u in range(UN):
                rows, sem_g, sem_c = bufs[u % 3]
                prows, psem_g, psem_c = bufs[(u + 2) % 3]

                pltpu.make_async_copy(x_hbm.at[lst(si_all, i, u)], rows,
                                      sem_g).wait()

                def wprev():
                    pltpu.make_async_copy(prows, out_sh.at[lst(di_all, 0, 0)],
                                          psem_c).wait()
                if u == 0:
                    @pl.when(i >= 1)
                    def _():
                        wprev()
                else:
                    wprev()

                if u < UN - 2:
                    pltpu.async_copy(x_hbm.at[lst(si_all, i, u + 2)], prows,
                                     psem_g)
                else:
                    @pl.when(i + 1 < n_iters)
                    def _():
                        pltpu.async_copy(
                            x_hbm.at[lst(si_all, i + 1, u + 2 - UN)], prows,
                            psem_g)
                compute(i, u, rows)
                pltpu.async_copy(rows, out_sh.at[lst(di_all, i, u)], sem_c,
                                 add=True)
            return 0
        lax.fori_loop(0, n_iters, it, 0)
        pltpu.make_async_copy(bufs[(UN - 1) % 3][0],
                              out_sh.at[lst(di_all, 0, 0)],
                              bufs[(UN - 1) % 3][2]).wait()

        plsc.subcore_barrier()
        pltpu.sync_copy(out_sh.at[pl.ds(s * sl, sl)],
                        out_hbm.at[c, pl.ds(s * sl, sl)])

    f = pl.kernel(
        body,
        out_type=jax.ShapeDtypeStruct((NC, NP, D), F32),
        mesh=mesh,
        compiler_params=pltpu.CompilerParams(needs_layout_passes=False),
        scratch_types=[
            pltpu.VMEM((nrow, 128), I32),
            pltpu.VMEM((nrow, 128), I32),
            pltpu.VMEM((nrow, 128), F32),
            pltpu.VMEM((C2, D), F32),
            pltpu.VMEM((C2, D), F32),
            pltpu.VMEM((C2, D), F32),
            pltpu.VMEM_SHARED((NP, D), F32),
            pltpu.SemaphoreType.DMA,
            pltpu.SemaphoreType.DMA,
            pltpu.SemaphoreType.DMA,
            pltpu.SemaphoreType.DMA,
            pltpu.SemaphoreType.DMA,
            pltpu.SemaphoreType.DMA,
        ],
    )
    return f(xp, w2, src2, dst2)


def _layer(xp, rn, src2, dst2, beta):
    NP = xp.shape[0]
    bp = jnp.stack([jnp.full((16,), beta, F32),
                    jnp.full((16,), jnp.abs(beta), F32)])
    w2, dp = _sc_pass1(xp, rn, bp, src2, dst2)
    op = _sc_pass2(xp, w2, src2, dst2)
    h, rn_next = _combine_tc(op, dp)
    return h, rn_next.reshape(NP)


def kernel(x, edge_index, beta1, beta2):
    N, D = x.shape
    E = edge_index.shape[1]
    NP = ((N + BLK - 1) // BLK) * BLK
    Et = E + N
    PT = 10752
    Et_pad = -(-Et // (NW * PT)) * NW * PT

    loops = jnp.arange(N, dtype=I32)
    pad_e = Et_pad - Et
    pad_idx = jnp.full((pad_e,), N, I32)
    src_flat = jnp.concatenate([edge_index[0].astype(I32), loops, pad_idx])
    dst_flat = jnp.concatenate([edge_index[1].astype(I32), loops, pad_idx])
    src2 = src_flat.reshape(NW, -1, CH)
    dst2 = dst_flat.reshape(NW, -1, CH)
    xp = jnp.zeros((NP, D), F32).at[:N].set(x)

    rn1 = _rnorm_tc(xp).reshape(NP)
    h1, rn2 = _layer(xp, rn1, src2, dst2, beta1.astype(F32))
    h2, _ = _layer(h1, rn2, src2, dst2, beta2.astype(F32))
    return h2[:N]


# speedup vs baseline: 5.1128x; 1.4204x over previous
import functools

import jax
import jax.numpy as jnp
from jax import lax
from jax.experimental import pallas as pl
from jax.experimental.pallas import tpu as pltpu
from jax.experimental.pallas import tpu_sc as plsc

F32 = jnp.float32
I32 = jnp.int32

NC = 2
TPS = 16
NW = NC * TPS
CH = 128
BLK = 1280


def _rnorm_tc(xp):
    NP, D = xp.shape

    def body(x_ref, rn_ref):
        xs = x_ref[...]
        ss = jnp.sum(xs * xs, axis=1)
        rn_ref[...] = (1.0 / jnp.maximum(jnp.sqrt(ss), 1e-12)).reshape(
            1, 8, BLK // 8)

    return pl.pallas_call(
        body,
        grid=(NP // BLK,),
        in_specs=[pl.BlockSpec((BLK, D), lambda i: (i, 0))],
        out_specs=pl.BlockSpec((1, 8, BLK // 8), lambda i: (i, 0, 0)),
        out_shape=jax.ShapeDtypeStruct((NP // BLK, 8, BLK // 8), F32),
    )(xp)


def _combine_tc(op, dp):
    _, NP, D = op.shape

    def body(p_ref, d_ref, h_ref, rn_ref):
        v = p_ref[0] + p_ref[1]
        dn = d_ref[0] + d_ref[1]
        rdn = 1.0 / jnp.maximum(dn, 1e-16)
        h = jnp.maximum(v * rdn[:, None], 0.0)
        h_ref[...] = h
        ss = jnp.sum(h * h, axis=1)
        rn_ref[...] = (1.0 / jnp.maximum(jnp.sqrt(ss), 1e-12)).reshape(
            1, 8, BLK // 8)

    return pl.pallas_call(
        body,
        grid=(NP // BLK,),
        in_specs=[
            pl.BlockSpec((2, BLK, D), lambda i: (0, i, 0)),
            pl.BlockSpec((2, BLK), lambda i: (0, i)),
        ],
        out_specs=[
            pl.BlockSpec((BLK, D), lambda i: (i, 0)),
            pl.BlockSpec((1, 8, BLK // 8), lambda i: (i, 0, 0)),
        ],
        out_shape=[
            jax.ShapeDtypeStruct((NP, D), F32),
            jax.ShapeDtypeStruct((NP // BLK, 8, BLK // 8), F32),
        ],
    )(op, dp)


def _sc_pass1(xp, rnorm, bp, src2, dst2):
    NP, D = xp.shape
    n_chunks = src2.shape[1]
    sl = NP // TPS
    mesh = plsc.VectorSubcoreMesh(core_axis_name="c", subcore_axis_name="s")

    def body(x_hbm, rn_hbm, bp_hbm, src_hbm, dst_hbm, w_hbm, dnm_hbm,
             rn_v, bp_v, si_all, di_all, rows_s0, rows_d0, rows_s1, rows_d1,
             wt_v, zb_v, dnm_sh, sem_s0, sem_d0, sem_s1, sem_d1, sem_n):
        c = lax.axis_index("c")
        s = lax.axis_index("s")
        wid = s * NC + c
        pltpu.sync_copy(rn_hbm, rn_v)
        pltpu.sync_copy(bp_hbm, bp_v)
        pltpu.sync_copy(src_hbm.at[wid], si_all)
        pltpu.sync_copy(dst_hbm.at[wid], di_all)
        betav = bp_v[0]
        shiftv = bp_v[1]

        def zg(g, _):
            zb_v[pl.ds(g * 16, 16)] = jnp.zeros((16,), F32)
            return 0
        lax.fori_loop(0, CH // 16, zg, 0)

        def zc(j, _):
            pltpu.sync_copy(zb_v, dnm_sh.at[pl.ds(s * sl + j * CH, CH)])
            return 0
        lax.fori_loop(0, sl // CH, zc, 0)
        plsc.subcore_barrier()

        lanes16 = lax.broadcasted_iota(I32, (16,), 0)

        def issue(j, rows_s, rows_d, sem_s, sem_d):
            pltpu.async_copy(x_hbm.at[si_all.at[j]], rows_s, sem_s)
            pltpu.async_copy(x_hbm.at[di_all.at[j]], rows_d, sem_d)

        def wait(j, rows_s, rows_d, sem_s, sem_d):
            pltpu.make_async_copy(x_hbm.at[si_all.at[j]], rows_s, sem_s).wait()
            pltpu.make_async_copy(x_hbm.at[di_all.at[j]], rows_d, sem_d).wait()

        def compute(j, rows_s, rows_d):
            def grp(g, _):
                wt = jnp.zeros((16,), F32)
                for l in range(16):
                    e = g * 16 + l
                    acc = rows_s[e, pl.ds(0, 16)] * rows_d[e, pl.ds(0, 16)]
                    for k in range(1, D // 16):
                        acc = acc + (rows_s[e, pl.ds(k * 16, 16)] *
                                     rows_d[e, pl.ds(k * 16, 16)])
                    wt = wt + jnp.where(lanes16 == l, jnp.sum(acc), 0.0)
                rs = plsc.load_gather(rn_v, [si_all[j, pl.ds(g * 16, 16)]])
                rd = plsc.load_gather(rn_v, [di_all[j, pl.ds(g * 16, 16)]])
                wt_v[j, pl.ds(g * 16, 16)] = jnp.exp(
                    wt * rs * rd * betav - shiftv)
                return 0
            lax.fori_loop(0, CH // 16, grp, 0)
            pltpu.async_copy(wt_v.at[j], dnm_sh.at[di_all.at[j]], sem_n,
                             add=True)

            @pl.when(j >= 2)
            def _():
                pltpu.make_async_copy(wt_v.at[0], dnm_sh.at[di_all.at[0]],
                                      sem_n).wait()

        issue(0, rows_s0, rows_d0, sem_s0, sem_d0)
        issue(1, rows_s1, rows_d1, sem_s1, sem_d1)

        def pair(i, _):
            j0 = 2 * i
            j1 = 2 * i + 1
            wait(j0, rows_s0, rows_d0, sem_s0, sem_d0)
            compute(j0, rows_s0, rows_d0)

            @pl.when(j0 + 2 < n_chunks)
            def _():
                issue(j0 + 2, rows_s0, rows_d0, sem_s0, sem_d0)
            wait(j1, rows_s1, rows_d1, sem_s1, sem_d1)
            compute(j1, rows_s1, rows_d1)

            @pl.when(j1 + 2 < n_chunks)
            def _():
                issue(j1 + 2, rows_s1, rows_d1, sem_s1, sem_d1)
            return 0
        lax.fori_loop(0, n_chunks // 2, pair, 0)

        pltpu.sync_copy(wt_v, w_hbm.at[wid])
        pltpu.make_async_copy(wt_v.at[0], dnm_sh.at[di_all.at[0]],
                              sem_n).wait()
        pltpu.make_async_copy(wt_v.at[0], dnm_sh.at[di_all.at[0]],
                              sem_n).wait()
        plsc.subcore_barrier()
        pltpu.sync_copy(dnm_sh.at[pl.ds(s * sl, sl)],
                        dnm_hbm.at[c, pl.ds(s * sl, sl)])

    f = pl.kernel(
        body,
        out_type=(
            jax.ShapeDtypeStruct((NW, n_chunks, CH), F32),
            jax.ShapeDtypeStruct((NC, NP), F32),
        ),
        mesh=mesh,
        compiler_params=pltpu.CompilerParams(needs_layout_passes=False),
        scratch_types=[
            pltpu.VMEM((NP,), F32),
            pltpu.VMEM((2, 16), F32),
            pltpu.VMEM((n_chunks, CH), I32),
            pltpu.VMEM((n_chunks, CH), I32),
            pltpu.VMEM((CH, D), F32),
            pltpu.VMEM((CH, D), F32),
            pltpu.VMEM((CH, D), F32),
            pltpu.VMEM((CH, D), F32),
            pltpu.VMEM((n_chunks, CH), F32),
            pltpu.VMEM((CH,), F32),
            pltpu.VMEM_SHARED((NP,), F32),
            pltpu.SemaphoreType.DMA,
            pltpu.SemaphoreType.DMA,
            pltpu.SemaphoreType.DMA,
            pltpu.SemaphoreType.DMA,
            pltpu.SemaphoreType.DMA,
        ],
    )
    return f(xp, rnorm, bp, src2, dst2)


def _sc_pass2(xp, w2, src2, dst2):
    NP, D = xp.shape
    nrow = src2.shape[1]
    C2 = 32
    UN = 12
    n_chunks = nrow * 128 // C2
    n_iters = n_chunks // UN
    rpi = UN * C2 // 128
    sl = NP // TPS
    mesh = plsc.VectorSubcoreMesh(core_axis_name="c", subcore_axis_name="s")

    def body(x_hbm, w_hbm, src_hbm, dst_hbm, out_hbm,
             si_all, di_all, wt_all, rows0, rows1, rows2, out_sh,
             sem_g0, sem_g1, sem_g2, sem_c0, sem_c1, sem_c2):
        c = lax.axis_index("c")
        s = lax.axis_index("s")
        wid = s * NC + c
        pltpu.sync_copy(src_hbm.at[wid], si_all)
        pltpu.sync_copy(dst_hbm.at[wid], di_all)
        pltpu.sync_copy(w_hbm.at[wid], wt_all)

        def zrow(e, _):
            def zk(k, _):
                rows0[e, pl.ds(k * 16, 16)] = jnp.zeros((16,), F32)
                return 0
            lax.fori_loop(0, D // 16, zk, 0)
            return 0
        lax.fori_loop(0, C2, zrow, 0)

        def zc(j, _):
            pltpu.sync_copy(rows0, out_sh.at[pl.ds(s * sl + j * C2, C2)])
            return 0
        lax.fori_loop(0, sl // C2, zc, 0)
        plsc.subcore_barrier()

        bufs = ((rows0, sem_g0, sem_c0),
                (rows1, sem_g1, sem_c1),
                (rows2, sem_g2, sem_c2))

        def lst(tab, i, u):
            jj = i * rpi + (u * C2) // 128
            return tab.at[jj, pl.ds((u * C2) % 128, C2)]

        def compute(i, u, rows):
            jj = i * rpi + (u * C2) // 128
            o = (u * C2) % 128

            def grp(g, _):
                cvec = wt_all[jj, pl.ds(o + g * 16, 16)]
                for l in range(16):
                    cst = cvec[l]
                    e = g * 16 + l
                    for k in range(D // 16):
                        rows[e, pl.ds(k * 16, 16)] = (
                            rows[e, pl.ds(k * 16, 16)] * cst)
                return 0
            lax.fori_loop(0, C2 // 16, grp, 0)

        pltpu.async_copy(x_hbm.at[lst(si_all, 0, 0)], rows0, sem_g0)
        pltpu.async_copy(x_hbm.at[lst(si_all, 0, 1)], rows1, sem_g1)

        def it(i, _):
            for u in range(UN):
                rows, sem_g, sem_c = bufs[u % 3]
                prows, psem_g, psem_c = bufs[(u + 2) % 3]

                pltpu.make_async_copy(x_hbm.at[lst(si_all, i, u)], rows,
                                      sem_g).wait()

                def wprev():
                    pltpu.make_async_copy(prows, out_sh.at[lst(di_all, 0, 0)],
                                          psem_c).wait()
                if u == 0:
                    @pl.when(i >= 1)
                    def _():
                        wprev()
                else:
                    wprev()

                if u < UN - 2:
                    pltpu.async_copy(x_hbm.at[lst(si_all, i, u + 2)], prows,
                                     psem_g)
                else:
                    @pl.when(i + 1 < n_iters)
                    def _():
                        pltpu.async_copy(
                            x_hbm.at[lst(si_all, i + 1, u + 2 - UN)], prows,
                            psem_g)
                compute(i, u, rows)
                pltpu.async_copy(rows, out_sh.at[lst(di_all, i, u)], sem_c,
                                 add=True)
            return 0
        lax.fori_loop(0, n_iters, it, 0)
        pltpu.make_async_copy(bufs[(UN - 1) % 3][0],
                              out_sh.at[lst(di_all, 0, 0)],
                              bufs[(UN - 1) % 3][2]).wait()

        plsc.subcore_barrier()
        pltpu.sync_copy(out_sh.at[pl.ds(s * sl, sl)],
                        out_hbm.at[c, pl.ds(s * sl, sl)])

    f = pl.kernel(
        body,
        out_type=jax.ShapeDtypeStruct((NC, NP, D), F32),
        mesh=mesh,
        compiler_params=pltpu.CompilerParams(needs_layout_passes=False),
        scratch_types=[
            pltpu.VMEM((nrow, 128), I32),
            pltpu.VMEM((nrow, 128), I32),
            pltpu.VMEM((nrow, 128), F32),
            pltpu.VMEM((C2, D), F32),
            pltpu.VMEM((C2, D), F32),
            pltpu.VMEM((C2, D), F32),
            pltpu.VMEM_SHARED((NP, D), F32),
            pltpu.SemaphoreType.DMA,
            pltpu.SemaphoreType.DMA,
            pltpu.SemaphoreType.DMA,
            pltpu.SemaphoreType.DMA,
            pltpu.SemaphoreType.DMA,
            pltpu.SemaphoreType.DMA,
        ],
    )
    return f(xp, w2, src2, dst2)


def _layer(xp, rn, src2, dst2, beta):
    NP = xp.shape[0]
    bp = jnp.stack([jnp.full((16,), beta, F32),
                    jnp.full((16,), jnp.abs(beta), F32)])
    w2, dp = _sc_pass1(xp, rn, bp, src2, dst2)
    op = _sc_pass2(xp, w2, src2, dst2)
    h, rn_next = _combine_tc(op, dp)
    return h, rn_next.reshape(NP)


def kernel(x, edge_index, beta1, beta2):
    N, D = x.shape
    E = edge_index.shape[1]
    NP = ((N + BLK - 1) // BLK) * BLK
    Et = E + N
    PT = 10752
    Et_pad = -(-Et // (NW * PT)) * NW * PT

    loops = jnp.arange(N, dtype=I32)
    pad_e = Et_pad - Et
    pad_src = jnp.full((pad_e,), N, I32)
    pad_dst = N + jnp.arange(pad_e, dtype=I32) % (NP - N)
    src_flat = jnp.concatenate([edge_index[0].astype(I32), loops, pad_src])
    dst_flat = jnp.concatenate([edge_index[1].astype(I32), loops, pad_dst])
    src2 = src_flat.reshape(NW, -1, CH)
    dst2 = dst_flat.reshape(NW, -1, CH)
    xp = jnp.zeros((NP, D), F32).at[:N].set(x)

    rn1 = _rnorm_tc(xp).reshape(NP)
    h1, rn2 = _layer(xp, rn1, src2, dst2, beta1.astype(F32))
    h2, _ = _layer(h1, rn2, src2, dst2, beta2.astype(F32))
    return h2[:N]
